# baseline (speedup 1.0000x reference)
import functools

import jax
import jax.numpy as jnp
import numpy as np
from jax import lax
from jax.experimental import pallas as pl
from jax.experimental.pallas import tpu as pltpu
from jax.experimental.pallas import tpu_sc as plsc

NBC = 16
NROWS = 1600
NCOLS = 1000
HALF_ROWS = 800
CHUNK_ROWS = 40
NCHUNKS = 63
RW = 1024
RHALF = 24 * RW
HALF_OUT = 10240


def _col0(k):
    return 16 * k if k < 62 else 984


_GROUPS = ((0, 16), (16, 32), (32, 48), (48, NCHUNKS))

_mesh = plsc.VectorSubcoreMesh(core_axis_name="c", subcore_axis_name="s")


def _worker():
    c_idx = lax.axis_index("c")
    s_idx = lax.axis_index("s")
    pair = s_idx // 2
    j = s_idx % 2
    bc = c_idx * 8 + pair
    return bc, j


@functools.partial(
    pl.kernel,
    mesh=_mesh,
    out_type=jax.ShapeDtypeStruct((NBC * 2 * RHALF,), jnp.float32),
    scratch_types=[
        pltpu.VMEM((CHUNK_ROWS, NCOLS), jnp.float32),
        pltpu.VMEM((CHUNK_ROWS, NCOLS), jnp.float32),
        pltpu.VMEM((RHALF,), jnp.float32),
        pltpu.SemaphoreType.DMA,
        pltpu.SemaphoreType.DMA,
        pltpu.SemaphoreType.DMA,
        pltpu.SemaphoreType.DMA,
    ],
)
def _reduce_sc(x_hbm, r_hbm, in0, in1, r_my, sem0a, sem0b, sem1a, sem1b):
    bc, j = _worker()
    row_base = j * HALF_ROWS

    def dma(blk, buf, sems):
        base = row_base + blk * CHUNK_ROWS
        return (
            pltpu.make_async_copy(
                x_hbm.at[bc, pl.ds(base, 24), :],
                buf.at[pl.ds(0, 24)], sems[0]),
            pltpu.make_async_copy(
                x_hbm.at[bc, pl.ds(base + 24, 16), :],
                buf.at[pl.ds(24, 16)], sems[1]),
        )

    def accumulate(blk, buf):
        for g0, g1 in _GROUPS:
            nk = g1 - g0
            zeros = tuple(jnp.zeros((16,), jnp.float32) for _ in range(nk))

            def r_body(r, acc, _g0=g0, _nk=nk):
                return tuple(
                    acc[i] + buf[r, pl.ds(_col0(_g0 + i), 16)]
                    for i in range(_nk))

            acc = plsc.parallel_loop(
                0, CHUNK_ROWS, unroll=4, carry=zeros)(r_body)
            for i in range(nk):
                r_my[pl.ds(blk * RW + _col0(g0 + i), 16)] = acc[i]

    sems0 = (sem0a, sem0b)
    sems1 = (sem1a, sem1b)

    def start(blk, buf, sems):
        for cp in dma(blk, buf, sems):
            cp.start()

    def wait(blk, buf, sems):
        for cp in dma(blk, buf, sems):
            cp.wait()

    start(0, in0, sems0)
    start(1, in1, sems1)

    def m_body(m, carry):
        wait(2 * m, in0, sems0)
        accumulate(2 * m, in0)

        @pl.when(m < 9)
        def _():
            start(2 * m + 2, in0, sems0)

        wait(2 * m + 1, in1, sems1)
        accumulate(2 * m + 1, in1)

        @pl.when(m < 9)
        def _():
            start(2 * m + 3, in1, sems1)

        return carry

    lax.fori_loop(0, 10, m_body, 0)

    pltpu.sync_copy(r_my, r_hbm.at[pl.ds((bc * 2 + j) * RHALF, RHALF)])


@functools.partial(
    pl.kernel,
    mesh=_mesh,
    compiler_params=pltpu.CompilerParams(
        use_tc_tiling_on_sc=False, needs_layout_passes=False),
    out_type=jax.ShapeDtypeStruct((NBC, 2 * HALF_OUT), jnp.float32),
    scratch_types=[
        pltpu.VMEM((20 * RW,), jnp.float32),
        pltpu.VMEM((20 * RW,), jnp.float32),
        pltpu.VMEM((HALF_OUT,), jnp.float32),
        pltpu.SemaphoreType.DMA,
        pltpu.SemaphoreType.DMA,
    ],
)
def _stitch_sc(r_hbm, out_hbm, r_a, r_b, out_v, sem_a, sem_b):
    bc, j = _worker()
    t0 = j * HALF_OUT

    cp_a = pltpu.make_async_copy(
        r_hbm.at[pl.ds((bc * 2) * RHALF, 20 * RW)], r_a, sem_a)
    cp_b = pltpu.make_async_copy(
        r_hbm.at[pl.ds((bc * 2 + 1) * RHALF, 20 * RW)], r_b, sem_b)
    cp_a.start()
    cp_b.start()
    cp_a.wait()
    cp_b.wait()

    zero = jnp.zeros((16,), jnp.float32)
    iot = lax.iota(jnp.int32, 16)
    scale = jnp.float32(0.025)

    def sample_idx(i):
        t = t0 + 16 * i + iot
        tf = t.astype(jnp.float32)
        s = (tf * jnp.float32(0.05) + jnp.float32(1e-3)).astype(jnp.int32)
        u = t - 20 * s
        return s, u

    def masked_chunk(i):
        s, u = sample_idx(i)
        u_row = lax.shift_left(u, 10)
        g1 = plsc.load_gather(r_a, [u_row + jnp.minimum(s, 999)])
        v1 = jnp.where(s <= 999, g1, zero)
        col2 = jnp.minimum(jnp.maximum(s - 1, 0), 999)
        g2 = plsc.load_gather(r_b, [u_row + col2])
        v2 = jnp.where((s >= 1) & (s <= 1000), g2, zero)
        out_v[pl.ds(16 * i, 16)] = (v1 + v2) * scale

    def interior_chunk(i):
        s, u = sample_idx(i)
        idx1 = lax.shift_left(u, 10) + s
        g1 = plsc.load_gather(r_a, [idx1])
        g2 = plsc.load_gather(r_b, [idx1 - 1])
        out_v[pl.ds(16 * i, 16)] = (g1 + g2) * scale

    def interior_loop(lo, hi):
        plsc.parallel_loop(lo, hi, unroll=8)(interior_chunk)

    @pl.when(j == 0)
    def _():
        masked_chunk(0)
        masked_chunk(1)
        interior_loop(2, 640)

    @pl.when(j == 1)
    def _():
        interior_loop(0, 610)
        masked_chunk(610)
        masked_chunk(611)

        def zbody(i):
            out_v[pl.ds(16 * i, 16)] = zero

        plsc.parallel_loop(612, 640)(zbody)

    pltpu.sync_copy(out_v, out_hbm.at[bc, pl.ds(t0, HALF_OUT)])


@jax.jit
def kernel(est_source):
    x = est_source.reshape(NBC, NROWS, NCOLS)
    r = _reduce_sc(x)
    full = _stitch_sc(r)
    return full[:, :20020].reshape(8, 2, 20020)


# speedup vs baseline: 1.0160x; 1.0160x over previous
import functools

import jax
import jax.numpy as jnp
import numpy as np
from jax import lax
from jax.experimental import pallas as pl
from jax.experimental.pallas import tpu as pltpu
from jax.experimental.pallas import tpu_sc as plsc

NBC = 16
NROWS = 1600
NCOLS = 1000
HALF_ROWS = 800
CHUNK_ROWS = 40
NCHUNKS = 63
RW = 1024
RHALF = 24 * RW
HALF_OUT = 10240


def _col0(k):
    return 16 * k if k < 62 else 984


_GROUPS = ((0, 16), (16, 32), (32, 48), (48, NCHUNKS))

_mesh = plsc.VectorSubcoreMesh(core_axis_name="c", subcore_axis_name="s")


def _worker():
    c_idx = lax.axis_index("c")
    s_idx = lax.axis_index("s")
    pair = s_idx // 2
    j = s_idx % 2
    bc = c_idx * 8 + pair
    return bc, j


@functools.partial(
    pl.kernel,
    mesh=_mesh,
    out_type=jax.ShapeDtypeStruct((NBC * 2 * RHALF,), jnp.float32),
    scratch_types=[
        pltpu.VMEM((CHUNK_ROWS, NCOLS), jnp.float32),
        pltpu.VMEM((CHUNK_ROWS, NCOLS), jnp.float32),
        pltpu.VMEM((RHALF,), jnp.float32),
        pltpu.SemaphoreType.DMA,
        pltpu.SemaphoreType.DMA,
        pltpu.SemaphoreType.DMA,
        pltpu.SemaphoreType.DMA,
    ],
)
def _reduce_sc(x_hbm, r_hbm, in0, in1, r_my, sem0a, sem0b, sem1a, sem1b):
    bc, j = _worker()
    row_base = j * HALF_ROWS

    def dma(blk, buf, sems):
        return (
            pltpu.make_async_copy(
                x_hbm.at[bc, pl.ds(row_base + blk * CHUNK_ROWS, CHUNK_ROWS),
                         :],
                buf, sems[0]),
        )

    def accumulate(blk, buf):
        for g0, g1 in _GROUPS:
            nk = g1 - g0
            zeros = tuple(jnp.zeros((16,), jnp.float32) for _ in range(nk))

            def r_body(r, acc, _g0=g0, _nk=nk):
                return tuple(
                    acc[i] + buf[r, pl.ds(_col0(_g0 + i), 16)]
                    for i in range(_nk))

            acc = plsc.parallel_loop(
                0, CHUNK_ROWS, unroll=4, carry=zeros)(r_body)
            for i in range(nk):
                r_my[pl.ds(blk * RW + _col0(g0 + i), 16)] = acc[i]

    sems0 = (sem0a, sem0b)
    sems1 = (sem1a, sem1b)

    def start(blk, buf, sems):
        for cp in dma(blk, buf, sems):
            cp.start()

    def wait(blk, buf, sems):
        for cp in dma(blk, buf, sems):
            cp.wait()

    start(0, in0, sems0)
    start(1, in1, sems1)

    def m_body(m, carry):
        wait(2 * m, in0, sems0)
        accumulate(2 * m, in0)

        @pl.when(m < 9)
        def _():
            start(2 * m + 2, in0, sems0)

        wait(2 * m + 1, in1, sems1)
        accumulate(2 * m + 1, in1)

        @pl.when(m < 9)
        def _():
            start(2 * m + 3, in1, sems1)

        return carry

    lax.fori_loop(0, 10, m_body, 0)

    pltpu.sync_copy(r_my, r_hbm.at[pl.ds((bc * 2 + j) * RHALF, RHALF)])


@functools.partial(
    pl.kernel,
    mesh=_mesh,
    compiler_params=pltpu.CompilerParams(
        use_tc_tiling_on_sc=False, needs_layout_passes=False),
    out_type=jax.ShapeDtypeStruct((NBC, 2 * HALF_OUT), jnp.float32),
    scratch_types=[
        pltpu.VMEM((20 * RW,), jnp.float32),
        pltpu.VMEM((20 * RW,), jnp.float32),
        pltpu.VMEM((HALF_OUT,), jnp.float32),
        pltpu.SemaphoreType.DMA,
        pltpu.SemaphoreType.DMA,
    ],
)
def _stitch_sc(r_hbm, out_hbm, r_a, r_b, out_v, sem_a, sem_b):
    bc, j = _worker()
    t0 = j * HALF_OUT

    cp_a = pltpu.make_async_copy(
        r_hbm.at[pl.ds((bc * 2) * RHALF, 20 * RW)], r_a, sem_a)
    cp_b = pltpu.make_async_copy(
        r_hbm.at[pl.ds((bc * 2 + 1) * RHALF, 20 * RW)], r_b, sem_b)
    cp_a.start()
    cp_b.start()
    cp_a.wait()
    cp_b.wait()

    zero = jnp.zeros((16,), jnp.float32)
    iot = lax.iota(jnp.int32, 16)
    scale = jnp.float32(0.025)

    def sample_idx(i):
        t = t0 + 16 * i + iot
        tf = t.astype(jnp.float32)
        s = (tf * jnp.float32(0.05) + jnp.float32(1e-3)).astype(jnp.int32)
        u = t - 20 * s
        return s, u

    def masked_chunk(i):
        s, u = sample_idx(i)
        u_row = lax.shift_left(u, 10)
        g1 = plsc.load_gather(r_a, [u_row + jnp.minimum(s, 999)])
        v1 = jnp.where(s <= 999, g1, zero)
        col2 = jnp.minimum(jnp.maximum(s - 1, 0), 999)
        g2 = plsc.load_gather(r_b, [u_row + col2])
        v2 = jnp.where((s >= 1) & (s <= 1000), g2, zero)
        out_v[pl.ds(16 * i, 16)] = (v1 + v2) * scale

    consts = []
    for c in range(5):
        off = 16 * c + iot
        offf = off.astype(jnp.float32)
        dsub = (offf * jnp.float32(0.05)
                + jnp.float32(1e-3)).astype(jnp.int32)
        u = off - 20 * dsub
        consts.append(lax.shift_left(u, 10) + dsub)

    def macro_loop(b_lo, b_hi, s_off):
        def body(b):
            s4 = jnp.broadcast_to(s_off + 4 * b, (16,))
            for c in range(5):
                idx1 = consts[c] + s4
                g1 = plsc.load_gather(r_a, [idx1])
                g2 = plsc.load_gather(r_b, [idx1 - 1])
                out_v[pl.ds(16 * (5 * b + c), 16)] = (g1 + g2) * scale

        plsc.parallel_loop(b_lo, b_hi, unroll=2)(body)

    @pl.when(j == 0)
    def _():
        for i in range(5):
            masked_chunk(i)
        macro_loop(1, 128, 0)

    @pl.when(j == 1)
    def _():
        macro_loop(0, 122, 512)
        masked_chunk(610)
        masked_chunk(611)

        def zbody(i):
            out_v[pl.ds(16 * i, 16)] = zero

        plsc.parallel_loop(612, 640)(zbody)

    pltpu.sync_copy(out_v, out_hbm.at[bc, pl.ds(t0, HALF_OUT)])


@jax.jit
def kernel(est_source):
    x = est_source.reshape(NBC, NROWS, NCOLS)
    r = _reduce_sc(x)
    full = _stitch_sc(r)
    return full[:, :20020].reshape(8, 2, 20020)
